# baseline (speedup 1.0000x reference)
import functools

import jax
import jax.numpy as jnp
from jax import lax
from jax.experimental import pallas as pl
from jax.experimental.pallas import tpu as pltpu
from jax.experimental.pallas import tpu_sc as plsc

N, C, IH, IW = 4, 96, 384, 384
H, W = 384, 384
HW = H * W
NPIX = N * HW
NWORKERS = 32
PPW = NPIX // NWORKERS
CHUNK = 128
NCHUNKS = PPW // CHUNK
L = 16


def _sc_sample(zt, gxy):
    mesh = plsc.VectorSubcoreMesh(core_axis_name="c", subcore_axis_name="s")

    @functools.partial(
        pl.kernel,
        mesh=mesh,
        compiler_params=pltpu.CompilerParams(
            use_tc_tiling_on_sc=False, needs_layout_passes=False),
        out_type=jax.ShapeDtypeStruct((N, C, HW), jnp.float32),
        scratch_types=[
            pltpu.VMEM((2, CHUNK), jnp.float32),
            pltpu.VMEM((2, 4, CHUNK), jnp.int32),
            pltpu.VMEM((2, 4, CHUNK), jnp.float32),
            [pltpu.VMEM((CHUNK, C), jnp.float32)] * 8,
            pltpu.VMEM((2, C, CHUNK), jnp.float32),
            pltpu.SemaphoreType.DMA((2, 4)),
            pltpu.SemaphoreType.DMA((2,)),
        ],
    )
    def k(zt_hbm, gxy_hbm, out_hbm,
          gxy_v, idx4, w4, rows8, out_v, gsem, osem):
        rows = [rows8[0:4], rows8[4:8]]
        wid = lax.axis_index("s") * 2 + lax.axis_index("c")
        n = wid // (NWORKERS // N)
        nbase = n * HW
        hw0 = (wid % (NWORKERS // N)) * PPW

        def stage(g, b):
            pltpu.sync_copy(gxy_hbm.at[wid * NCHUNKS + g], gxy_v)
            for t in range(CHUNK // L):
                s = pl.ds(t * L, L)
                x = gxy_v[0, s]
                y = gxy_v[1, s]
                ix = (x + 1.0) * 0.5 * (IW - 1)
                iy = (y + 1.0) * 0.5 * (IH - 1)
                ix0 = ix.astype(jnp.int32)
                iy0 = iy.astype(jnp.int32)
                ix0f = ix0.astype(jnp.float32)
                iy0f = iy0.astype(jnp.float32)
                wx1 = (ix0f + 1.0) - ix
                wx0 = ix - ix0f
                wy1 = (iy0f + 1.0) - iy
                wy0 = iy - iy0f
                ix0c = jnp.minimum(jnp.maximum(ix0, 0), IW - 1)
                iy0c = jnp.minimum(jnp.maximum(iy0, 0), IH - 1)
                ix1c = jnp.minimum(ix0c + 1, IW - 1)
                iy1c = jnp.minimum(iy0c + 1, IH - 1)
                r0 = nbase + iy0c * IW
                r1 = nbase + iy1c * IW
                idx4[b, 0, s] = r0 + ix0c
                idx4[b, 1, s] = r0 + ix1c
                idx4[b, 2, s] = r1 + ix0c
                idx4[b, 3, s] = r1 + ix1c
                w4[b, 0, s] = wx1 * wy1
                w4[b, 1, s] = wx0 * wy1
                w4[b, 2, s] = wx1 * wy0
                w4[b, 3, s] = wx0 * wy0
            for kk in range(4):
                pltpu.async_copy(zt_hbm.at[idx4.at[b, kk]],
                                 rows[b][kk], gsem.at[b, kk])

        def wait_gathers(b):
            for kk in range(4):
                pltpu.make_async_copy(zt_hbm.at[idx4.at[b, kk]],
                                      rows[b][kk], gsem.at[b, kk]).wait()

        def out_slice(g):
            return out_hbm.at[n, :, pl.ds(hw0 + g * CHUNK, CHUNK)]

        def blend(g, b):
            def t_body(t, _):
                s = pl.ds(t * L, L)
                pix = lax.iota(jnp.int32, L) + t * L
                anw = w4[b, 0, s]
                ane = w4[b, 1, s]
                asw = w4[b, 2, s]
                ase = w4[b, 3, s]
                for cc in range(C):
                    ccv = jnp.full((L,), cc, dtype=jnp.int32)
                    vnw = plsc.load_gather(rows[b][0], [pix, ccv])
                    vne = plsc.load_gather(rows[b][1], [pix, ccv])
                    vsw = plsc.load_gather(rows[b][2], [pix, ccv])
                    vse = plsc.load_gather(rows[b][3], [pix, ccv])
                    out_v[b, cc, s] = (anw * vnw + ane * vne) + (
                        asw * vsw + ase * vse)
                return 0

            lax.fori_loop(0, CHUNK // L, t_body, 0)
            pltpu.async_copy(out_v.at[b], out_slice(g), osem.at[b])

        def wait_out(g, b):
            pltpu.make_async_copy(out_v.at[b], out_slice(g), osem.at[b]).wait()

        stage(0, 0)

        def pair_body(gg, _):
            for b in range(2):
                g = gg * 2 + b
                nb = 1 - b
                @pl.when(g + 1 < NCHUNKS)
                def _():
                    stage(g + 1, nb)
                wait_gathers(b)
                @pl.when(g >= 2)
                def _():
                    wait_out(g - 2, b)
                blend(g, b)
            return 0

        lax.fori_loop(0, NCHUNKS // 2, pair_body, 0)
        wait_out(NCHUNKS - 2, 0)
        wait_out(NCHUNKS - 1, 1)

    return k(zt, gxy)


def kernel(z, grid):
    zt = jnp.transpose(z, (0, 2, 3, 1)).reshape(NPIX, C)
    gx = grid[..., 0].reshape(-1, CHUNK)
    gy = grid[..., 1].reshape(-1, CHUNK)
    gxy = jnp.stack([gx, gy], axis=1)
    out = _sc_sample(zt, gxy)
    return out.reshape(N, C, H, W)


# speedup vs baseline: 3.0245x; 3.0245x over previous
import functools

import jax
import jax.numpy as jnp
from jax import lax
from jax.experimental import pallas as pl
from jax.experimental.pallas import tpu as pltpu
from jax.experimental.pallas import tpu_sc as plsc

N, C, IH, IW = 4, 96, 384, 384
H, W = 384, 384
HW = H * W
NPIX = N * HW
NWORKERS = 32
PPW = NPIX // NWORKERS
CHUNK = 128
OPITCH = CHUNK + 1
NCHUNKS = PPW // CHUNK
L = 16


def _sc_sample(zt, gxy):
    mesh = plsc.VectorSubcoreMesh(core_axis_name="c", subcore_axis_name="s")

    @functools.partial(
        pl.kernel,
        mesh=mesh,
        compiler_params=pltpu.CompilerParams(
            use_tc_tiling_on_sc=False, needs_layout_passes=False),
        out_type=jax.ShapeDtypeStruct((N, C, HW), jnp.float32),
        scratch_types=[
            pltpu.VMEM((2, CHUNK), jnp.float32),
            pltpu.VMEM((2, 4, CHUNK), jnp.int32),
            pltpu.VMEM((2, 4, CHUNK), jnp.float32),
            [pltpu.VMEM((CHUNK, C), jnp.float32)] * 8,
            [pltpu.VMEM((C, OPITCH), jnp.float32)] * 2,
            pltpu.SemaphoreType.DMA((2, 4)),
            pltpu.SemaphoreType.DMA((2,)),
        ],
    )
    def k(zt_hbm, gxy_hbm, out_hbm,
          gxy_v, idx4, w4, rows8, outv, gsem, osem):
        rows = [rows8[0:4], rows8[4:8]]
        wid = lax.axis_index("s") * 2 + lax.axis_index("c")
        n = wid // (NWORKERS // N)
        nbase = n * HW
        hw0 = (wid % (NWORKERS // N)) * PPW
        chvs = [lax.iota(jnp.int32, L) + j * L for j in range(C // L)]

        def stage(g, b):
            pltpu.sync_copy(gxy_hbm.at[wid * NCHUNKS + g], gxy_v)
            for t in range(CHUNK // L):
                s = pl.ds(t * L, L)
                x = gxy_v[0, s]
                y = gxy_v[1, s]
                ix = (x + 1.0) * 0.5 * (IW - 1)
                iy = (y + 1.0) * 0.5 * (IH - 1)
                ix0 = ix.astype(jnp.int32)
                iy0 = iy.astype(jnp.int32)
                ix0f = ix0.astype(jnp.float32)
                iy0f = iy0.astype(jnp.float32)
                wx1 = (ix0f + 1.0) - ix
                wx0 = ix - ix0f
                wy1 = (iy0f + 1.0) - iy
                wy0 = iy - iy0f
                ix0c = jnp.minimum(jnp.maximum(ix0, 0), IW - 1)
                iy0c = jnp.minimum(jnp.maximum(iy0, 0), IH - 1)
                ix1c = jnp.minimum(ix0c + 1, IW - 1)
                iy1c = jnp.minimum(iy0c + 1, IH - 1)
                r0 = nbase + iy0c * IW
                r1 = nbase + iy1c * IW
                idx4[b, 0, s] = r0 + ix0c
                idx4[b, 1, s] = r0 + ix1c
                idx4[b, 2, s] = r1 + ix0c
                idx4[b, 3, s] = r1 + ix1c
                w4[b, 0, s] = wx1 * wy1
                w4[b, 1, s] = wx0 * wy1
                w4[b, 2, s] = wx1 * wy0
                w4[b, 3, s] = wx0 * wy0
            for kk in range(4):
                pltpu.async_copy(zt_hbm.at[idx4.at[b, kk]],
                                 rows[b][kk], gsem.at[b, kk])

        def wait_gathers(b):
            for kk in range(4):
                pltpu.make_async_copy(zt_hbm.at[idx4.at[b, kk]],
                                      rows[b][kk], gsem.at[b, kk]).wait()

        def out_copy_args(g, b):
            return (outv[b].at[:, pl.ds(0, CHUNK)],
                    out_hbm.at[n, :, pl.ds(hw0 + g * CHUNK, CHUNK)],
                    osem.at[b])

        def blend(g, b):
            rnw, rne, rsw, rse = rows[b]
            ov = outv[b]

            def t_body(t, _):
                s = pl.ds(t * L, L)
                av = w4[b, 0, s]
                bv = w4[b, 1, s]
                cv = w4[b, 2, s]
                dv = w4[b, 3, s]
                for lane in range(L):
                    i = t * L + lane
                    wa = av[lane]
                    wb = bv[lane]
                    wc = cv[lane]
                    wd = dv[lane]
                    iv = jnp.zeros((L,), jnp.int32) + i
                    for j in range(C // L):
                        cs = pl.ds(j * L, L)
                        res = (wa * rnw[i, cs] + wb * rne[i, cs]) + (
                            wc * rsw[i, cs] + wd * rse[i, cs])
                        plsc.store_scatter(ov, [chvs[j], iv], res)
                return 0

            lax.fori_loop(0, CHUNK // L, t_body, 0)
            pltpu.async_copy(*out_copy_args(g, b))

        def wait_out(g, b):
            pltpu.make_async_copy(*out_copy_args(g, b)).wait()

        stage(0, 0)

        def pair_body(gg, _):
            for b in range(2):
                g = gg * 2 + b
                nb = 1 - b
                @pl.when(g + 1 < NCHUNKS)
                def _():
                    stage(g + 1, nb)
                wait_gathers(b)
                @pl.when(g >= 2)
                def _():
                    wait_out(g - 2, b)
                blend(g, b)
            return 0

        lax.fori_loop(0, NCHUNKS // 2, pair_body, 0)
        wait_out(NCHUNKS - 2, 0)
        wait_out(NCHUNKS - 1, 1)

    return k(zt, gxy)


def kernel(z, grid):
    zt = jnp.transpose(z, (0, 2, 3, 1)).reshape(NPIX, C)
    gx = grid[..., 0].reshape(-1, CHUNK)
    gy = grid[..., 1].reshape(-1, CHUNK)
    gxy = jnp.stack([gx, gy], axis=1)
    out = _sc_sample(zt, gxy)
    return out.reshape(N, C, H, W)


# speedup vs baseline: 3.1792x; 1.0512x over previous
import functools

import jax
import jax.numpy as jnp
from jax import lax
from jax.experimental import pallas as pl
from jax.experimental.pallas import tpu as pltpu
from jax.experimental.pallas import tpu_sc as plsc

N, C, IH, IW = 4, 96, 384, 384
H, W = 384, 384
HW = H * W
NPIX = N * HW
NWORKERS = 32
PPW = NPIX // NWORKERS
CHUNK = 128
OPITCH = CHUNK + 1
NCHUNKS = PPW // CHUNK
NPAIRS = NCHUNKS // 2
L = 16


def _sc_sample(zt, gxy):
    mesh = plsc.VectorSubcoreMesh(core_axis_name="c", subcore_axis_name="s")

    @functools.partial(
        pl.kernel,
        mesh=mesh,
        compiler_params=pltpu.CompilerParams(
            use_tc_tiling_on_sc=False, needs_layout_passes=False),
        out_type=jax.ShapeDtypeStruct((N, C, HW), jnp.float32),
        scratch_types=[
            [pltpu.VMEM((2, 2, CHUNK), jnp.float32)] * 2,
            [pltpu.VMEM((4 * CHUNK,), jnp.int32)] * 2,
            pltpu.VMEM((2, 4, CHUNK), jnp.float32),
            [pltpu.VMEM((4 * CHUNK, C), jnp.float32)] * 2,
            [pltpu.VMEM((C, OPITCH), jnp.float32)] * 2,
            pltpu.SemaphoreType.DMA((2,)),
            pltpu.SemaphoreType.DMA((2,)),
            pltpu.SemaphoreType.DMA((2,)),
        ],
    )
    def k(zt_hbm, gxy_hbm, out_hbm,
          gxyv, idxs, w4, rowss, outv, psem, gsem, osem):
        wid = lax.axis_index("s") * 2 + lax.axis_index("c")
        n = wid // (NWORKERS // N)
        nbase = n * HW
        hw0 = (wid % (NWORKERS // N)) * PPW
        cpair0 = wid * NCHUNKS
        chvs = [lax.iota(jnp.int32, L) + j * L for j in range(C // L)]

        def gxy_copy_args(gg, p):
            return (gxy_hbm.at[pl.ds(cpair0 + gg * 2, 2)], gxyv[p], psem.at[p])

        def gather_copy_args(b):
            return (zt_hbm.at[idxs[b]], rowss[b], gsem.at[b])

        def out_copy_args(g, b):
            return (outv[b].at[:, pl.ds(0, CHUNK)],
                    out_hbm.at[n, :, pl.ds(hw0 + g * CHUNK, CHUNK)],
                    osem.at[b])

        def stage(half, p, b):
            idx = idxs[b]
            for t in range(CHUNK // L):
                s = pl.ds(t * L, L)
                x = gxyv[p][half, 0, s]
                y = gxyv[p][half, 1, s]
                ix = (x + 1.0) * 0.5 * (IW - 1)
                iy = (y + 1.0) * 0.5 * (IH - 1)
                ix0 = ix.astype(jnp.int32)
                iy0 = iy.astype(jnp.int32)
                ix0f = ix0.astype(jnp.float32)
                iy0f = iy0.astype(jnp.float32)
                wx1 = (ix0f + 1.0) - ix
                wx0 = ix - ix0f
                wy1 = (iy0f + 1.0) - iy
                wy0 = iy - iy0f
                ix0c = jnp.minimum(jnp.maximum(ix0, 0), IW - 1)
                iy0c = jnp.minimum(jnp.maximum(iy0, 0), IH - 1)
                ix1c = jnp.minimum(ix0c + 1, IW - 1)
                iy1c = jnp.minimum(iy0c + 1, IH - 1)
                r0 = nbase + iy0c * IW
                r1 = nbase + iy1c * IW
                idx[pl.ds(0 * CHUNK + t * L, L)] = r0 + ix0c
                idx[pl.ds(1 * CHUNK + t * L, L)] = r0 + ix1c
                idx[pl.ds(2 * CHUNK + t * L, L)] = r1 + ix0c
                idx[pl.ds(3 * CHUNK + t * L, L)] = r1 + ix1c
                w4[b, 0, s] = wx1 * wy1
                w4[b, 1, s] = wx0 * wy1
                w4[b, 2, s] = wx1 * wy0
                w4[b, 3, s] = wx0 * wy0
            pltpu.async_copy(*gather_copy_args(b))

        def blend(g, b):
            rows = rowss[b]
            ov = outv[b]

            def t_body(t, _):
                s = pl.ds(t * L, L)
                av = w4[b, 0, s]
                bv = w4[b, 1, s]
                cv = w4[b, 2, s]
                dv = w4[b, 3, s]
                for lane in range(L):
                    i = t * L + lane
                    wa = av[lane]
                    wb = bv[lane]
                    wc = cv[lane]
                    wd = dv[lane]
                    iv = jnp.zeros((L,), jnp.int32) + i
                    for j in range(C // L):
                        cs = pl.ds(j * L, L)
                        res = (wa * rows[0 * CHUNK + i, cs]
                               + wb * rows[1 * CHUNK + i, cs]) + (
                            wc * rows[2 * CHUNK + i, cs]
                            + wd * rows[3 * CHUNK + i, cs])
                        plsc.store_scatter(ov, [chvs[j], iv], res)
                return 0

            lax.fori_loop(0, CHUNK // L, t_body, 0)
            pltpu.async_copy(*out_copy_args(g, b))

        pltpu.async_copy(*gxy_copy_args(0, 0))
        pltpu.async_copy(*gxy_copy_args(1, 1))
        pltpu.make_async_copy(*gxy_copy_args(0, 0)).wait()
        stage(0, 0, 0)

        def quad_body(gg2, _):
            for pp in range(2):
                gg = gg2 * 2 + pp
                for b in range(2):
                    g = gg * 2 + b
                    if b == 0:
                        stage(1, pp, 1)
                    else:
                        @pl.when(g + 1 < NCHUNKS)
                        def _():
                            pltpu.make_async_copy(
                                *gxy_copy_args(gg + 1, 1 - pp)).wait()

                            @pl.when(gg + 2 < NPAIRS)
                            def _():
                                pltpu.async_copy(*gxy_copy_args(gg + 2, pp))

                            stage(0, 1 - pp, 0)

                    pltpu.make_async_copy(*gather_copy_args(b)).wait()
                    @pl.when(g >= 2)
                    def _():
                        pltpu.make_async_copy(*out_copy_args(g - 2, b)).wait()
                    blend(g, b)
            return 0

        lax.fori_loop(0, NPAIRS // 2, quad_body, 0)
        pltpu.make_async_copy(*out_copy_args(NCHUNKS - 2, 0)).wait()
        pltpu.make_async_copy(*out_copy_args(NCHUNKS - 1, 1)).wait()

    return k(zt, gxy)


def kernel(z, grid):
    zt = jnp.transpose(z, (0, 2, 3, 1)).reshape(NPIX, C)
    gx = grid[..., 0].reshape(-1, CHUNK)
    gy = grid[..., 1].reshape(-1, CHUNK)
    gxy = jnp.stack([gx, gy], axis=1)
    out = _sc_sample(zt, gxy)
    return out.reshape(N, C, H, W)


# speedup vs baseline: 3.6187x; 1.1382x over previous
import functools

import jax
import jax.numpy as jnp
from jax import lax
from jax.experimental import pallas as pl
from jax.experimental.pallas import tpu as pltpu
from jax.experimental.pallas import tpu_sc as plsc

N, C, IH, IW = 4, 96, 384, 384
H, W = 384, 384
HW = H * W
NPIX = N * HW
NWORKERS = 32
PPW = NPIX // NWORKERS
CHUNK = 128
OPITCH = CHUNK + 1
NCHUNKS = PPW // CHUNK
NPAIRS = NCHUNKS // 2
L = 16


def _sc_sample(zt, gxy):
    mesh = plsc.VectorSubcoreMesh(core_axis_name="c", subcore_axis_name="s")

    @functools.partial(
        pl.kernel,
        mesh=mesh,
        compiler_params=pltpu.CompilerParams(
            use_tc_tiling_on_sc=False, needs_layout_passes=False),
        out_type=jax.ShapeDtypeStruct((N, C, HW), jnp.float32),
        scratch_types=[
            [pltpu.VMEM((2, 2, CHUNK), jnp.float32)] * 2,
            [pltpu.VMEM((4 * CHUNK,), jnp.int32)] * 2,
            pltpu.VMEM((2, 4, CHUNK), jnp.float32),
            [pltpu.VMEM((4 * CHUNK, C), jnp.bfloat16)] * 2,
            [pltpu.VMEM((C, OPITCH), jnp.float32)] * 2,
            pltpu.SemaphoreType.DMA((2,)),
            pltpu.SemaphoreType.DMA((2,)),
            pltpu.SemaphoreType.DMA((2,)),
        ],
    )
    def k(zt_hbm, gxy_hbm, out_hbm,
          gxyv, idxs, w4, rowss, outv, psem, gsem, osem):
        wid = lax.axis_index("s") * 2 + lax.axis_index("c")
        n = wid // (NWORKERS // N)
        nbase = n * HW
        hw0 = (wid % (NWORKERS // N)) * PPW
        cpair0 = wid * NCHUNKS
        chv_e = [lax.iota(jnp.int32, L) * 2 + jj * 2 * L for jj in range(C // (2 * L))]
        chv_o = [lax.iota(jnp.int32, L) * 2 + 1 + jj * 2 * L for jj in range(C // (2 * L))]

        def gxy_copy_args(gg, p):
            return (gxy_hbm.at[pl.ds(cpair0 + gg * 2, 2)], gxyv[p], psem.at[p])

        def gather_copy_args(b):
            return (zt_hbm.at[idxs[b]], rowss[b], gsem.at[b])

        def out_copy_args(g, b):
            return (outv[b].at[:, pl.ds(0, CHUNK)],
                    out_hbm.at[n, :, pl.ds(hw0 + g * CHUNK, CHUNK)],
                    osem.at[b])

        def stage(half, p, b):
            idx = idxs[b]
            for t in range(CHUNK // L):
                s = pl.ds(t * L, L)
                x = gxyv[p][half, 0, s]
                y = gxyv[p][half, 1, s]
                ix = (x + 1.0) * 0.5 * (IW - 1)
                iy = (y + 1.0) * 0.5 * (IH - 1)
                ix0 = ix.astype(jnp.int32)
                iy0 = iy.astype(jnp.int32)
                ix0f = ix0.astype(jnp.float32)
                iy0f = iy0.astype(jnp.float32)
                wx1 = (ix0f + 1.0) - ix
                wx0 = ix - ix0f
                wy1 = (iy0f + 1.0) - iy
                wy0 = iy - iy0f
                ix0c = jnp.minimum(jnp.maximum(ix0, 0), IW - 1)
                iy0c = jnp.minimum(jnp.maximum(iy0, 0), IH - 1)
                ix1c = jnp.minimum(ix0c + 1, IW - 1)
                iy1c = jnp.minimum(iy0c + 1, IH - 1)
                r0 = nbase + iy0c * IW
                r1 = nbase + iy1c * IW
                idx[pl.ds(0 * CHUNK + t * L, L)] = r0 + ix0c
                idx[pl.ds(1 * CHUNK + t * L, L)] = r0 + ix1c
                idx[pl.ds(2 * CHUNK + t * L, L)] = r1 + ix0c
                idx[pl.ds(3 * CHUNK + t * L, L)] = r1 + ix1c
                w4[b, 0, s] = wx1 * wy1
                w4[b, 1, s] = wx0 * wy1
                w4[b, 2, s] = wx1 * wy0
                w4[b, 3, s] = wx0 * wy0
            pltpu.async_copy(*gather_copy_args(b))

        def blend(g, b):
            rows = rowss[b]
            ov = outv[b]

            def t_body(t, _):
                s = pl.ds(t * L, L)
                av = w4[b, 0, s]
                bv = w4[b, 1, s]
                cv = w4[b, 2, s]
                dv = w4[b, 3, s]
                for lane in range(L):
                    i = t * L + lane
                    wa = av[lane]
                    wb = bv[lane]
                    wc = cv[lane]
                    wd = dv[lane]
                    iv = jnp.zeros((L,), jnp.int32) + i
                    for jj in range(C // (2 * L)):
                        cs = pl.ds(jj * 2 * L, 2 * L)
                        enw, onw = plsc.unpack(
                            rows[0 * CHUNK + i, cs],
                            format=plsc.PackFormat.INTERLEAVED)
                        ene, one = plsc.unpack(
                            rows[1 * CHUNK + i, cs],
                            format=plsc.PackFormat.INTERLEAVED)
                        esw, osw = plsc.unpack(
                            rows[2 * CHUNK + i, cs],
                            format=plsc.PackFormat.INTERLEAVED)
                        ese, ose = plsc.unpack(
                            rows[3 * CHUNK + i, cs],
                            format=plsc.PackFormat.INTERLEAVED)
                        res_e = (wa * enw + wb * ene) + (wc * esw + wd * ese)
                        res_o = (wa * onw + wb * one) + (wc * osw + wd * ose)
                        plsc.store_scatter(ov, [chv_e[jj], iv], res_e)
                        plsc.store_scatter(ov, [chv_o[jj], iv], res_o)
                return 0

            lax.fori_loop(0, CHUNK // L, t_body, 0)
            pltpu.async_copy(*out_copy_args(g, b))

        pltpu.async_copy(*gxy_copy_args(0, 0))
        pltpu.async_copy(*gxy_copy_args(1, 1))
        pltpu.make_async_copy(*gxy_copy_args(0, 0)).wait()
        stage(0, 0, 0)

        def quad_body(gg2, _):
            for pp in range(2):
                gg = gg2 * 2 + pp
                for b in range(2):
                    g = gg * 2 + b
                    if b == 0:
                        stage(1, pp, 1)
                    else:
                        @pl.when(g + 1 < NCHUNKS)
                        def _():
                            pltpu.make_async_copy(
                                *gxy_copy_args(gg + 1, 1 - pp)).wait()

                            @pl.when(gg + 2 < NPAIRS)
                            def _():
                                pltpu.async_copy(*gxy_copy_args(gg + 2, pp))

                            stage(0, 1 - pp, 0)

                    pltpu.make_async_copy(*gather_copy_args(b)).wait()
                    @pl.when(g >= 2)
                    def _():
                        pltpu.make_async_copy(*out_copy_args(g - 2, b)).wait()
                    blend(g, b)
            return 0

        lax.fori_loop(0, NPAIRS // 2, quad_body, 0)
        pltpu.make_async_copy(*out_copy_args(NCHUNKS - 2, 0)).wait()
        pltpu.make_async_copy(*out_copy_args(NCHUNKS - 1, 1)).wait()

    return k(zt, gxy)


def kernel(z, grid):
    zt = jnp.transpose(z, (0, 2, 3, 1)).reshape(NPIX, C).astype(jnp.bfloat16)
    gx = grid[..., 0].reshape(-1, CHUNK)
    gy = grid[..., 1].reshape(-1, CHUNK)
    gxy = jnp.stack([gx, gy], axis=1)
    out = _sc_sample(zt, gxy)
    return out.reshape(N, C, H, W)
